# baseline (speedup 1.0000x reference)
import functools

import jax
import jax.numpy as jnp
from jax import lax
from jax.experimental import pallas as pl
from jax.experimental.pallas import tpu as pltpu
from jax.experimental.pallas import tpu_sc as plsc

B = 4096
N = 16384
L = 16
NC = 2
NS = 16
NW = NC * NS
ROWS_PER_W = B // NW
KU = 8
BU = 8
JBITS = 14
JMASK = (1 << JBITS) - 1


@functools.cache
def _build(nnzp: int):
    mesh = plsc.VectorSubcoreMesh(
        core_axis_name="c", subcore_axis_name="s", num_cores=NC, num_subcores=NS
    )

    @functools.partial(
        pl.kernel,
        out_type=jax.ShapeDtypeStruct((B, N), jnp.float32),
        mesh=mesh,
        compiler_params=pltpu.CompilerParams(needs_layout_passes=False),
        scratch_types=[
            pltpu.VMEM((nnzp,), jnp.int32),
            pltpu.VMEM((nnzp,), jnp.float32),
            pltpu.VMEM((N,), jnp.float32),
            pltpu.VMEM((N,), jnp.float32),
            pltpu.VMEM((N,), jnp.float32),
            pltpu.VMEM((N,), jnp.float32),
            pltpu.VMEM((N,), jnp.float32),
            pltpu.SemaphoreType.DMA,
            pltpu.SemaphoreType.DMA,
            pltpu.SemaphoreType.DMA,
            pltpu.SemaphoreType.DMA,
        ],
    )
    def sc_kernel(inputs_hbm, w3_hbm, b_hbm, vel_hbm, packed_hbm, out_hbm,
                  packed, vals, biasv, x0, x1, y0, y1,
                  sx0, sx1, sy0, sy1):
        wid = lax.axis_index("s") * NC + lax.axis_index("c")
        row0 = wid * ROWS_PER_W

        pltpu.sync_copy(packed_hbm, packed)
        pltpu.sync_copy(w3_hbm, y0.at[pl.ds(0, nnzp)])
        pltpu.sync_copy(vel_hbm, x0)
        pltpu.sync_copy(b_hbm, biasv)

        @plsc.parallel_loop(0, nnzp // L, unroll=KU)
        def val_body(t):
            o = t * L
            pk = packed[pl.ds(o, L)]
            jv = lax.bitwise_and(pk, JMASK)
            g = plsc.load_gather(x0, [jv])
            vals[pl.ds(o, L)] = y0[pl.ds(o, L)] * g

        def bias_init(ybuf):
            @plsc.parallel_loop(0, N // L, unroll=BU)
            def bias_body(i):
                o = i * L
                ybuf[pl.ds(o, L)] = biasv[pl.ds(o, L)]

        def k_loop(xbuf, ybuf):
            @plsc.parallel_loop(0, nnzp // L, unroll=KU)
            def k_body(t):
                o = t * L
                pk = packed[pl.ds(o, L)]
                jv = lax.bitwise_and(pk, JMASK)
                iv = lax.shift_right_logical(pk, JBITS)
                g = plsc.load_gather(xbuf, [jv])
                plsc.addupdate_scatter(ybuf, [iv], vals[pl.ds(o, L)] * g)

        pltpu.async_copy(inputs_hbm.at[row0], x0, sx0)

        def pair_body(p, c):
            ra = row0 + 2 * p
            rb = ra + 1
            pltpu.make_async_copy(inputs_hbm.at[ra], x0, sx0).wait()
            pltpu.async_copy(inputs_hbm.at[rb], x1, sx1)

            @pl.when(p > 0)
            def _():
                pltpu.make_async_copy(y0, out_hbm.at[ra - 2], sy0).wait()

            bias_init(y0)
            k_loop(x0, y0)
            pltpu.async_copy(y0, out_hbm.at[ra], sy0)

            pltpu.make_async_copy(inputs_hbm.at[rb], x1, sx1).wait()

            @pl.when(p < ROWS_PER_W // 2 - 1)
            def _():
                pltpu.async_copy(inputs_hbm.at[ra + 2], x0, sx0)

            @pl.when(p > 0)
            def _():
                pltpu.make_async_copy(y1, out_hbm.at[rb - 2], sy1).wait()

            bias_init(y1)
            k_loop(x1, y1)
            pltpu.async_copy(y1, out_hbm.at[rb], sy1)
            return c

        lax.fori_loop(0, ROWS_PER_W // 2, pair_body, 0)
        last = row0 + ROWS_PER_W
        pltpu.make_async_copy(y0, out_hbm.at[last - 2], sy0).wait()
        pltpu.make_async_copy(y1, out_hbm.at[last - 1], sy1).wait()

    return sc_kernel


def kernel(inputs, W3, b, velocity, I, J):
    nnz = W3.shape[0]
    chunk = L * KU
    nnzp = ((nnz + chunk - 1) // chunk) * chunk
    pad = nnzp - nnz
    I32 = I.astype(jnp.int32)
    J32 = J.astype(jnp.int32)
    packed = jnp.left_shift(I32, JBITS) | J32
    pad_packed = jnp.left_shift(jnp.arange(pad, dtype=jnp.int32), JBITS)
    packed = jnp.concatenate([packed, pad_packed])
    W3p = jnp.concatenate([W3, jnp.zeros((pad,), jnp.float32)])
    groups = nnzp // L
    packed = packed.reshape(L, groups).T.reshape(-1)
    W3p = W3p.reshape(L, groups).T.reshape(-1)
    return _build(nnzp)(inputs, W3p, b, velocity, packed)


# speedup vs baseline: 1.0877x; 1.0877x over previous
import functools

import jax
import jax.numpy as jnp
from jax import lax
from jax.experimental import pallas as pl
from jax.experimental.pallas import tpu as pltpu
from jax.experimental.pallas import tpu_sc as plsc

B = 4096
N = 16384
L = 16
NC = 2
NS = 16
NW = NC * NS
ROWS_PER_W = B // NW
KU = 8
BU = 8
JBITS = 14
JMASK = (1 << JBITS) - 1


@functools.cache
def _build(nnzp: int):
    mesh = plsc.VectorSubcoreMesh(
        core_axis_name="c", subcore_axis_name="s", num_cores=NC, num_subcores=NS
    )

    @functools.partial(
        pl.kernel,
        out_type=jax.ShapeDtypeStruct((B, N), jnp.float32),
        mesh=mesh,
        compiler_params=pltpu.CompilerParams(needs_layout_passes=False),
        scratch_types=[
            pltpu.VMEM((nnzp,), jnp.int32),
            pltpu.VMEM((nnzp,), jnp.int32),
            pltpu.VMEM((nnzp,), jnp.float32),
            pltpu.VMEM((N,), jnp.float32),
            pltpu.VMEM((N,), jnp.float32),
            pltpu.VMEM((N,), jnp.float32),
            pltpu.VMEM((N,), jnp.float32),
            pltpu.VMEM((N,), jnp.float32),
            pltpu.SemaphoreType.DMA,
            pltpu.SemaphoreType.DMA,
            pltpu.SemaphoreType.DMA,
            pltpu.SemaphoreType.DMA,
        ],
    )
    def sc_kernel(inputs_hbm, w3_hbm, b_hbm, vel_hbm, packed_hbm, perm_hbm,
                  out_hbm,
                  packed, perm, vals, biasv, x0, x1, y0, y1,
                  sx0, sx1, sy0, sy1):
        wid = lax.axis_index("s") * NC + lax.axis_index("c")
        row0 = wid * ROWS_PER_W

        pltpu.sync_copy(packed_hbm, packed)
        pltpu.sync_copy(perm_hbm, perm)
        pltpu.sync_copy(w3_hbm, y0.at[pl.ds(0, nnzp)])
        pltpu.sync_copy(vel_hbm, x0)
        pltpu.sync_copy(b_hbm, biasv)

        @plsc.parallel_loop(0, nnzp // L, unroll=KU)
        def val_body(t):
            o = t * L
            pk = packed[pl.ds(o, L)]
            jv = lax.bitwise_and(pk, JMASK)
            g = plsc.load_gather(x0, [jv])
            vals[pl.ds(o, L)] = y0[pl.ds(o, L)] * g

        @plsc.parallel_loop(0, nnzp // L, unroll=KU)
        def perm_body(t):
            o = t * L
            pv = perm[pl.ds(o, L)]
            pk2 = plsc.load_gather(packed, [pv])
            vv2 = plsc.load_gather(vals, [pv])
            y1[pl.ds(o, L)] = plsc.bitcast(pk2, jnp.float32)
            x1[pl.ds(o, L)] = vv2

        @plsc.parallel_loop(0, nnzp // L, unroll=KU)
        def copyback_body(t):
            o = t * L
            packed[pl.ds(o, L)] = plsc.bitcast(y1[pl.ds(o, L)], jnp.int32)
            vals[pl.ds(o, L)] = x1[pl.ds(o, L)]

        def bias_init(ybuf):
            @plsc.parallel_loop(0, N // L, unroll=BU)
            def bias_body(i):
                o = i * L
                ybuf[pl.ds(o, L)] = biasv[pl.ds(o, L)]

        def k_loop(xbuf, ybuf):
            @plsc.parallel_loop(0, nnzp // L, unroll=KU)
            def k_body(t):
                o = t * L
                pk = packed[pl.ds(o, L)]
                jv = lax.bitwise_and(pk, JMASK)
                iv = lax.shift_right_logical(pk, JBITS)
                g = plsc.load_gather(xbuf, [jv])
                plsc.addupdate_scatter(ybuf, [iv], vals[pl.ds(o, L)] * g)

        pltpu.async_copy(inputs_hbm.at[row0], x0, sx0)

        def pair_body(p, c):
            ra = row0 + 2 * p
            rb = ra + 1
            pltpu.make_async_copy(inputs_hbm.at[ra], x0, sx0).wait()
            pltpu.async_copy(inputs_hbm.at[rb], x1, sx1)

            @pl.when(p > 0)
            def _():
                pltpu.make_async_copy(y0, out_hbm.at[ra - 2], sy0).wait()

            bias_init(y0)
            k_loop(x0, y0)
            pltpu.async_copy(y0, out_hbm.at[ra], sy0)

            pltpu.make_async_copy(inputs_hbm.at[rb], x1, sx1).wait()

            @pl.when(p < ROWS_PER_W // 2 - 1)
            def _():
                pltpu.async_copy(inputs_hbm.at[ra + 2], x0, sx0)

            @pl.when(p > 0)
            def _():
                pltpu.make_async_copy(y1, out_hbm.at[rb - 2], sy1).wait()

            bias_init(y1)
            k_loop(x1, y1)
            pltpu.async_copy(y1, out_hbm.at[rb], sy1)
            return c

        lax.fori_loop(0, ROWS_PER_W // 2, pair_body, 0)
        last = row0 + ROWS_PER_W
        pltpu.make_async_copy(y0, out_hbm.at[last - 2], sy0).wait()
        pltpu.make_async_copy(y1, out_hbm.at[last - 1], sy1).wait()

    return sc_kernel


def kernel(inputs, W3, b, velocity, I, J):
    nnz = W3.shape[0]
    chunk = L * KU
    nnzp = ((nnz + chunk - 1) // chunk) * chunk
    pad = nnzp - nnz
    I32 = I.astype(jnp.int32)
    J32 = J.astype(jnp.int32)
    packed = jnp.left_shift(I32, JBITS) | J32
    pad_packed = jnp.left_shift(jnp.arange(pad, dtype=jnp.int32), JBITS)
    packed = jnp.concatenate([packed, pad_packed])
    W3p = jnp.concatenate([W3, jnp.zeros((pad,), jnp.float32)])
    ar = jnp.arange(nnz, dtype=jnp.int32)
    first = jnp.concatenate(
        [jnp.ones((1,), jnp.bool_), I32[1:] != I32[:-1]])
    seg_base = lax.cummax(jnp.where(first, ar, 0))
    rank = ar - seg_base
    perm = jnp.argsort(rank, stable=True).astype(jnp.int32)
    perm = jnp.concatenate([perm, jnp.arange(nnz, nnzp, dtype=jnp.int32)])
    return _build(nnzp)(inputs, W3p, b, velocity, packed, perm)


# speedup vs baseline: 1.0947x; 1.0065x over previous
import functools

import jax
import jax.numpy as jnp
from jax import lax
from jax.experimental import pallas as pl
from jax.experimental.pallas import tpu as pltpu
from jax.experimental.pallas import tpu_sc as plsc

B = 4096
N = 16384
L = 16
NC = 2
NS = 16
NW = NC * NS
ROWS_PER_W = B // NW
KU = 8
BU = 8
JBITS = 14
JMASK = (1 << JBITS) - 1


@functools.cache
def _build(nnzp: int):
    mesh = plsc.VectorSubcoreMesh(
        core_axis_name="c", subcore_axis_name="s", num_cores=NC, num_subcores=NS
    )

    @functools.partial(
        pl.kernel,
        out_type=jax.ShapeDtypeStruct((B, N), jnp.float32),
        mesh=mesh,
        compiler_params=pltpu.CompilerParams(needs_layout_passes=False),
        scratch_types=[
            pltpu.VMEM((nnzp,), jnp.int32),
            pltpu.VMEM((nnzp,), jnp.int32),
            pltpu.VMEM((nnzp,), jnp.float32),
            pltpu.VMEM((N,), jnp.float32),
            pltpu.VMEM((N,), jnp.float32),
            pltpu.VMEM((N,), jnp.float32),
            pltpu.VMEM((N,), jnp.float32),
            pltpu.VMEM((N,), jnp.float32),
            pltpu.SemaphoreType.DMA,
            pltpu.SemaphoreType.DMA,
            pltpu.SemaphoreType.DMA,
            pltpu.SemaphoreType.DMA,
        ],
    )
    def sc_kernel(inputs_hbm, w3_hbm, b_hbm, vel_hbm, packed_hbm, perm_hbm,
                  out_hbm,
                  packed, perm, vals, biasv, x0, x1, y0, y1,
                  sx0, sx1, sy0, sy1):
        wid = lax.axis_index("s") * NC + lax.axis_index("c")
        row0 = wid * ROWS_PER_W

        pltpu.async_copy(packed_hbm, packed, sx0)
        pltpu.async_copy(perm_hbm, perm, sx0)
        pltpu.async_copy(w3_hbm, y0.at[pl.ds(0, nnzp)], sx0)
        pltpu.async_copy(vel_hbm, x0, sx0)
        pltpu.async_copy(b_hbm, biasv, sx0)
        pltpu.make_async_copy(packed_hbm, packed, sx0).wait()
        pltpu.make_async_copy(perm_hbm, perm, sx0).wait()
        pltpu.make_async_copy(w3_hbm, y0.at[pl.ds(0, nnzp)], sx0).wait()
        pltpu.make_async_copy(vel_hbm, x0, sx0).wait()
        pltpu.make_async_copy(b_hbm, biasv, sx0).wait()

        @plsc.parallel_loop(0, nnzp // L, unroll=KU)
        def perm_body(t):
            o = t * L
            pv = perm[pl.ds(o, L)]
            pk2 = plsc.load_gather(packed, [pv])
            w = plsc.load_gather(y0, [pv])
            g = plsc.load_gather(x0, [lax.bitwise_and(pk2, JMASK)])
            y1[pl.ds(o, L)] = plsc.bitcast(pk2, jnp.float32)
            vals[pl.ds(o, L)] = w * g

        @plsc.parallel_loop(0, nnzp // L, unroll=KU)
        def copyback_body(t):
            o = t * L
            packed[pl.ds(o, L)] = plsc.bitcast(y1[pl.ds(o, L)], jnp.int32)

        def bias_init(ybuf):
            @plsc.parallel_loop(0, N // L, unroll=BU)
            def bias_body(i):
                o = i * L
                ybuf[pl.ds(o, L)] = biasv[pl.ds(o, L)]

        def k_loop(xbuf, ybuf):
            @plsc.parallel_loop(0, nnzp // L, unroll=KU)
            def k_body(t):
                o = t * L
                pk = packed[pl.ds(o, L)]
                jv = lax.bitwise_and(pk, JMASK)
                iv = lax.shift_right_logical(pk, JBITS)
                g = plsc.load_gather(xbuf, [jv])
                plsc.addupdate_scatter(ybuf, [iv], vals[pl.ds(o, L)] * g)

        pltpu.async_copy(inputs_hbm.at[row0], x0, sx0)

        def pair_body(p, c):
            ra = row0 + 2 * p
            rb = ra + 1
            pltpu.make_async_copy(inputs_hbm.at[ra], x0, sx0).wait()
            pltpu.async_copy(inputs_hbm.at[rb], x1, sx1)

            @pl.when(p > 0)
            def _():
                pltpu.make_async_copy(y0, out_hbm.at[ra - 2], sy0).wait()

            bias_init(y0)
            k_loop(x0, y0)
            pltpu.async_copy(y0, out_hbm.at[ra], sy0)

            pltpu.make_async_copy(inputs_hbm.at[rb], x1, sx1).wait()

            @pl.when(p < ROWS_PER_W // 2 - 1)
            def _():
                pltpu.async_copy(inputs_hbm.at[ra + 2], x0, sx0)

            @pl.when(p > 0)
            def _():
                pltpu.make_async_copy(y1, out_hbm.at[rb - 2], sy1).wait()

            bias_init(y1)
            k_loop(x1, y1)
            pltpu.async_copy(y1, out_hbm.at[rb], sy1)
            return c

        lax.fori_loop(0, ROWS_PER_W // 2, pair_body, 0)
        last = row0 + ROWS_PER_W
        pltpu.make_async_copy(y0, out_hbm.at[last - 2], sy0).wait()
        pltpu.make_async_copy(y1, out_hbm.at[last - 1], sy1).wait()

    return sc_kernel


def kernel(inputs, W3, b, velocity, I, J):
    nnz = W3.shape[0]
    chunk = L * KU
    nnzp = ((nnz + chunk - 1) // chunk) * chunk
    pad = nnzp - nnz
    I32 = I.astype(jnp.int32)
    J32 = J.astype(jnp.int32)
    packed = jnp.left_shift(I32, JBITS) | J32
    pad_packed = jnp.left_shift(jnp.arange(pad, dtype=jnp.int32), JBITS)
    packed = jnp.concatenate([packed, pad_packed])
    W3p = jnp.concatenate([W3, jnp.zeros((pad,), jnp.float32)])
    ar = jnp.arange(nnz, dtype=jnp.int32)
    first = jnp.concatenate(
        [jnp.ones((1,), jnp.bool_), I32[1:] != I32[:-1]])
    seg_base = lax.cummax(jnp.where(first, ar, 0))
    rank = ar - seg_base
    perm = jnp.argsort(rank, stable=True).astype(jnp.int32)
    perm = jnp.concatenate([perm, jnp.arange(nnz, nnzp, dtype=jnp.int32)])
    return _build(nnzp)(inputs, W3p, b, velocity, packed, perm)
